# baseline (speedup 1.0000x reference)
import jax
import jax.numpy as jnp
from jax.experimental import pallas as pl
from jax.experimental.pallas import tpu as pltpu

_E = 64
_K = 8
_ALPHA = 0.01


def _route_tile(x, wt):
    logits = jnp.dot(x, wt, preferred_element_type=jnp.float32)

    m = jnp.max(logits, axis=1, keepdims=True)
    ex = jnp.exp(logits - m)
    z = jnp.sum(ex, axis=1, keepdims=True)
    scores = (ex / z).T

    iota_f = jax.lax.broadcasted_iota(jnp.int32, scores.shape, 0).astype(
        jnp.float32)
    sbits = jax.lax.bitcast_convert_type(scores, jnp.int32)
    work = sbits
    mks = []
    idxs = []
    for _ in range(_K):
        mk = jnp.max(work, axis=0, keepdims=True)
        eq = work == mk
        idxs.append(jnp.sum(jnp.where(eq, iota_f, 0.0), axis=0,
                            keepdims=True))
        work = jnp.where(eq, jnp.int32(-(2**31)), work)
        mks.append(mk)

    mkcat = jnp.concatenate(mks, axis=0)
    ti = jnp.concatenate(idxs, axis=0).astype(jnp.int32)
    tw = jax.lax.bitcast_convert_type(mkcat, jnp.float32)
    tw = tw / (jnp.sum(tw, axis=0, keepdims=True) + 1e-9)

    hits = (sbits >= mks[-1]).astype(jnp.float32)
    ssum = jnp.sum(scores, axis=1, keepdims=True)
    cnt = jnp.sum(hits, axis=1, keepdims=True)
    return tw, ti, ssum, cnt


def _router_block(xa_ref, xb_ref, wt_ref, twa_ref, tia_ref, twb_ref,
                  tib_ref, aux_ref, ssum_ref, cnt_ref):
    i = pl.program_id(0)
    n = pl.num_programs(0)

    wt = wt_ref[...]
    twa, tia, ssum_a, cnt_a = _route_tile(xa_ref[...], wt)
    twb, tib, ssum_b, cnt_b = _route_tile(xb_ref[...], wt)

    twa_ref[...] = twa
    tia_ref[...] = tia
    twb_ref[...] = twb
    tib_ref[...] = tib

    block_ssum = ssum_a + ssum_b
    block_cnt = cnt_a + cnt_b

    @pl.when(i == 0)
    def _init():
        ssum_ref[...] = block_ssum
        cnt_ref[...] = block_cnt

    @pl.when(i > 0)
    def _acc():
        ssum_ref[...] += block_ssum
        cnt_ref[...] += block_cnt

    @pl.when(i == n - 1)
    def _finish():
        t_total = 2 * n * xa_ref.shape[0]
        scale = _ALPHA * _E / (float(t_total) * float(t_total) * _K)
        s = jnp.sum(ssum_ref[...] * cnt_ref[...], axis=0, keepdims=True)
        aux_ref[...] = s * scale


def kernel(x, W):
    bsz, seq, d = x.shape
    t = bsz * seq
    half = t // 2
    xf = x.reshape(t, d)
    xa = xf[:half]
    xb = xf[half:]
    wt = W.T

    blk = 512
    grid = (half // blk,)

    twa, tia, twb, tib, aux = pl.pallas_call(
        _router_block,
        grid=grid,
        in_specs=[
            pl.BlockSpec((blk, d), lambda i: (i, 0)),
            pl.BlockSpec((blk, d), lambda i: (i, 0)),
            pl.BlockSpec((d, _E), lambda i: (0, 0)),
        ],
        out_specs=[
            pl.BlockSpec((_K, blk), lambda i: (0, i)),
            pl.BlockSpec((_K, blk), lambda i: (0, i)),
            pl.BlockSpec((_K, blk), lambda i: (0, i)),
            pl.BlockSpec((_K, blk), lambda i: (0, i)),
            pl.BlockSpec((1, 1), lambda i: (0, 0)),
        ],
        out_shape=[
            jax.ShapeDtypeStruct((_K, half), jnp.float32),
            jax.ShapeDtypeStruct((_K, half), jnp.int32),
            jax.ShapeDtypeStruct((_K, half), jnp.float32),
            jax.ShapeDtypeStruct((_K, half), jnp.int32),
            jax.ShapeDtypeStruct((1, 1), jnp.float32),
        ],
        scratch_shapes=[
            pltpu.VMEM((_E, 1), jnp.float32),
            pltpu.VMEM((_E, 1), jnp.float32),
        ],
    )(xa, xb, wt)

    tw = jnp.concatenate([twa, twb], axis=1).T
    ti = jnp.concatenate([tia, tib], axis=1).T
    return tw, ti, aux[0, 0]


# speedup vs baseline: 2.8950x; 2.8950x over previous
import jax
import jax.numpy as jnp
from jax.experimental import pallas as pl
from jax.experimental.pallas import tpu as pltpu

_E = 64
_K = 8
_ALPHA = 0.01


def _router_block(x_ref, wt_ref, tw_ref, ti_ref, aux_ref, ssum_ref, cnt_ref):
    i = pl.program_id(0)
    n = pl.num_programs(0)

    x = x_ref[...]
    wt = wt_ref[...]
    logits = jnp.dot(x, wt, preferred_element_type=jnp.float32)

    m = jnp.max(logits, axis=1, keepdims=True)
    ex = jnp.exp(logits - m)
    z = jnp.sum(ex, axis=1, keepdims=True)
    scores = (ex / z).T

    iota_f = jax.lax.broadcasted_iota(jnp.int32, scores.shape, 0).astype(
        jnp.float32)
    sbits = jax.lax.bitcast_convert_type(scores, jnp.int32)
    work = sbits
    mks = []
    idxs = []
    for _ in range(_K):
        mk = jnp.max(work, axis=0, keepdims=True)
        eq = work == mk
        idxs.append(jnp.sum(jnp.where(eq, iota_f, 0.0), axis=0,
                            keepdims=True))
        work = jnp.where(eq, jnp.int32(-(2**31)), work)
        mks.append(mk)

    mkcat = jnp.concatenate(mks, axis=0)
    ti = jnp.concatenate(idxs, axis=0).astype(jnp.int32)
    tw = jax.lax.bitcast_convert_type(mkcat, jnp.float32)
    tw = tw / (jnp.sum(tw, axis=0, keepdims=True) + 1e-9)

    tw_ref[...] = tw
    ti_ref[...] = ti

    hits = (sbits >= mks[-1]).astype(jnp.float32)
    block_ssum = jnp.sum(scores, axis=1, keepdims=True)
    block_cnt = jnp.sum(hits, axis=1, keepdims=True)

    @pl.when(i == 0)
    def _init():
        ssum_ref[...] = block_ssum
        cnt_ref[...] = block_cnt

    @pl.when(i > 0)
    def _acc():
        ssum_ref[...] += block_ssum
        cnt_ref[...] += block_cnt

    @pl.when(i == n - 1)
    def _finish():
        t_total = n * x.shape[0]
        scale = _ALPHA * _E / (float(t_total) * float(t_total) * _K)
        s = jnp.sum(ssum_ref[...] * cnt_ref[...], axis=0, keepdims=True)
        aux_ref[...] = s * scale


def kernel(x, W):
    bsz, seq, d = x.shape
    t = bsz * seq
    xf = x.reshape(t, d)
    wt = W.T

    blk = 1024
    grid = (t // blk,)

    tw_kt, ti_kt, aux = pl.pallas_call(
        _router_block,
        grid=grid,
        in_specs=[
            pl.BlockSpec((blk, d), lambda i: (i, 0)),
            pl.BlockSpec((d, _E), lambda i: (0, 0)),
        ],
        out_specs=[
            pl.BlockSpec((_K, blk), lambda i: (0, i)),
            pl.BlockSpec((_K, blk), lambda i: (0, i)),
            pl.BlockSpec((1, 1), lambda i: (0, 0)),
        ],
        out_shape=[
            jax.ShapeDtypeStruct((_K, t), jnp.float32),
            jax.ShapeDtypeStruct((_K, t), jnp.int32),
            jax.ShapeDtypeStruct((1, 1), jnp.float32),
        ],
        scratch_shapes=[
            pltpu.VMEM((_E, 1), jnp.float32),
            pltpu.VMEM((_E, 1), jnp.float32),
        ],
        compiler_params=pltpu.CompilerParams(
            vmem_limit_bytes=120 * 1024 * 1024,
        ),
    )(xf, wt)

    return tw_kt.T, ti_kt.T, aux[0, 0]
